# baseline (speedup 1.0000x reference)
import functools

import jax
import jax.numpy as jnp
from jax import lax
from jax.experimental import pallas as pl
from jax.experimental.pallas import tpu as pltpu
from jax.experimental.pallas import tpu_sc as plsc

VOCAB = 100000
EMB = 128
HID = 256
B = 1024
T = 50

_NC = 2
_NS = 16
_NW = _NC * _NS
_N_IDX = B * T
_PER_W = _N_IDX // _NW
_CHUNK = 80
_N_CHUNK = _PER_W // _CHUNK


_NBUF = 4


def _sc_gather(table, idx2d):
    mesh = plsc.VectorSubcoreMesh(core_axis_name="c", subcore_axis_name="s")

    @functools.partial(
        pl.kernel,
        out_type=jax.ShapeDtypeStruct((_N_IDX, EMB), jnp.float32),
        mesh=mesh,
        scratch_types=[
            pltpu.VMEM((_N_CHUNK, _CHUNK), jnp.int32),
            [pltpu.VMEM((_CHUNK, EMB), jnp.float32) for _ in range(_NBUF)],
            [pltpu.SemaphoreType.DMA for _ in range(_NBUF)],
            [pltpu.SemaphoreType.DMA for _ in range(_NBUF)],
        ],
    )
    def gather_kernel(table_hbm, idx_hbm, out_hbm, idx_v, bufs, gsems, ssems):
        wid = lax.axis_index("s") * _NC + lax.axis_index("c")
        base = wid * _PER_W
        pltpu.sync_copy(idx_hbm.at[wid], idx_v)

        gathers = [None] * _N_CHUNK
        stores = [None] * _N_CHUNK

        def start_gather(j):
            b = j % _NBUF
            gathers[j] = pltpu.async_copy(
                table_hbm.at[idx_v.at[j]], bufs[b], gsems[b]
            )

        for j in range(_NBUF):
            start_gather(j)
        for j in range(_N_CHUNK):
            b = j % _NBUF
            gathers[j].wait()
            stores[j] = pltpu.async_copy(
                bufs[b], out_hbm.at[pl.ds(base + j * _CHUNK, _CHUNK)], ssems[b]
            )
            nxt = j + _NBUF
            if nxt < _N_CHUNK:
                stores[j].wait()
                start_gather(nxt)
        for j in range(_N_CHUNK - _NBUF, _N_CHUNK):
            stores[j].wait()

    return gather_kernel(table, idx2d)


def _lstm_step(emb_ref, w_ref, b_ref, h_out, c_out, xh_s, c_s):
    t = pl.program_id(0)

    @pl.when(t == 0)
    def _():
        xh_s[:, EMB:] = jnp.zeros((B, HID), jnp.bfloat16)
        c_s[...] = jnp.zeros_like(c_s)

    xh_s[:, :EMB] = emb_ref[0].astype(jnp.bfloat16)
    _NSPLIT = 2
    _RB = B // _NSPLIT
    for k in range(_NSPLIT):
        rs = pl.ds(k * _RB, _RB)
        gates = (
            jnp.dot(xh_s[rs, :], w_ref[...],
                    preferred_element_type=jnp.float32)
            + b_ref[...]
        )
        i = 0.5 * jnp.tanh(gates[:, 0 * HID : 1 * HID]) + 0.5
        f = 0.5 * jnp.tanh(gates[:, 1 * HID : 2 * HID]) + 0.5
        g = jnp.tanh(gates[:, 2 * HID : 3 * HID])
        o = 0.5 * jnp.tanh(gates[:, 3 * HID : 4 * HID]) + 0.5
        c_new = f * c_s[rs, :] + i * g
        h_new = o * jnp.tanh(c_new)
        c_s[rs, :] = c_new
        xh_s[rs, EMB:] = h_new.astype(jnp.bfloat16)

        @pl.when(t == T - 1)
        def _():
            h_out[rs, :] = h_new
            c_out[rs, :] = c_new


def _tc_lstm(emb, w_cat, bias):
    out_shape = [
        jax.ShapeDtypeStruct((B, HID), jnp.float32),
        jax.ShapeDtypeStruct((B, HID), jnp.float32),
    ]
    grid = (T,)
    return pl.pallas_call(
        _lstm_step,
        grid=grid,
        in_specs=[
            pl.BlockSpec((1, B, EMB), lambda t: (t, 0, 0)),
            pl.BlockSpec((EMB + HID, 4 * HID), lambda t: (0, 0)),
            pl.BlockSpec((1, 4 * HID), lambda t: (0, 0)),
        ],
        out_specs=[
            pl.BlockSpec((B, HID), lambda t: (0, 0)),
            pl.BlockSpec((B, HID), lambda t: (0, 0)),
        ],
        out_shape=out_shape,
        scratch_shapes=[
            pltpu.VMEM((B, EMB + HID), jnp.bfloat16),
            pltpu.VMEM((B, HID), jnp.float32),
        ],
    )(emb, w_cat, bias)


def kernel(x, embedding_matrix, W_ih, W_hh, b_ih, b_hh):
    idx3d = jnp.reshape(
        jnp.transpose(x).astype(jnp.int32), (_NW, _N_CHUNK, _CHUNK)
    )
    emb_flat = _sc_gather(embedding_matrix, idx3d)
    emb = jnp.reshape(emb_flat, (T, B, EMB))
    scale = jnp.concatenate([
        jnp.full((HID,), 0.5, jnp.float32),
        jnp.full((HID,), 0.5, jnp.float32),
        jnp.ones((HID,), jnp.float32),
        jnp.full((HID,), 0.5, jnp.float32),
    ])
    w_cat = (
        jnp.concatenate([jnp.transpose(W_ih), jnp.transpose(W_hh)], axis=0)
        * scale[None, :]
    ).astype(jnp.bfloat16)
    bias = jnp.reshape((b_ih + b_hh) * scale, (1, 4 * HID))
    h, c = _tc_lstm(emb, w_cat, bias)
    return (h[None, :, :], c[None, :, :])


# speedup vs baseline: 1.1252x; 1.1252x over previous
import functools

import jax
import jax.numpy as jnp
from jax import lax
from jax.experimental import pallas as pl
from jax.experimental.pallas import tpu as pltpu
from jax.experimental.pallas import tpu_sc as plsc

VOCAB = 100000
EMB = 128
HID = 256
B = 1024
T = 50

_NC = 2
_NS = 16
_NW = _NC * _NS
_N_IDX = B * T
_PER_W = _N_IDX // _NW
_CHUNK = 80
_N_CHUNK = _PER_W // _CHUNK


_NBUF = 4


def _sc_gather(table, idx2d):
    mesh = plsc.VectorSubcoreMesh(core_axis_name="c", subcore_axis_name="s")

    @functools.partial(
        pl.kernel,
        out_type=jax.ShapeDtypeStruct((_N_IDX, EMB), jnp.float32),
        mesh=mesh,
        scratch_types=[
            pltpu.VMEM((_N_CHUNK, _CHUNK), jnp.int32),
            [pltpu.VMEM((_CHUNK, EMB), jnp.float32) for _ in range(_NBUF)],
            [pltpu.SemaphoreType.DMA for _ in range(_NBUF)],
            [pltpu.SemaphoreType.DMA for _ in range(_NBUF)],
        ],
    )
    def gather_kernel(table_hbm, idx_hbm, out_hbm, idx_v, bufs, gsems, ssems):
        wid = lax.axis_index("s") * _NC + lax.axis_index("c")
        base = wid * _PER_W
        pltpu.sync_copy(idx_hbm.at[wid], idx_v)

        gathers = [None] * _N_CHUNK
        stores = [None] * _N_CHUNK

        def start_gather(j):
            b = j % _NBUF
            gathers[j] = pltpu.async_copy(
                table_hbm.at[idx_v.at[j]], bufs[b], gsems[b]
            )

        for j in range(_NBUF):
            start_gather(j)
        for j in range(_N_CHUNK):
            b = j % _NBUF
            gathers[j].wait()
            stores[j] = pltpu.async_copy(
                bufs[b], out_hbm.at[pl.ds(base + j * _CHUNK, _CHUNK)], ssems[b]
            )
            nxt = j + _NBUF
            if nxt < _N_CHUNK:
                stores[j].wait()
                start_gather(nxt)
        for j in range(_N_CHUNK - _NBUF, _N_CHUNK):
            stores[j].wait()

    return gather_kernel(table, idx2d)


def _lstm_step(emb_ref, w_ref, b_ref, h_out, c_out, xh_s, c_s):
    t = pl.program_id(0)

    @pl.when(t == 0)
    def _():
        xh_s[:, EMB:] = jnp.zeros((B, HID), jnp.bfloat16)
        c_s[...] = jnp.zeros_like(c_s)

    xh_s[:, :EMB] = emb_ref[0].astype(jnp.bfloat16)
    gates = (
        jnp.dot(xh_s[...], w_ref[...], preferred_element_type=jnp.float32)
        + b_ref[...]
    )
    i = 0.5 * jnp.tanh(gates[:, 0 * HID : 1 * HID]) + 0.5
    f = 0.5 * jnp.tanh(gates[:, 1 * HID : 2 * HID]) + 0.5
    g = jnp.tanh(gates[:, 2 * HID : 3 * HID])
    o = 0.5 * jnp.tanh(gates[:, 3 * HID : 4 * HID]) + 0.5
    c_new = f * c_s[...] + i * g
    h_new = o * jnp.tanh(c_new)
    c_s[...] = c_new
    xh_s[:, EMB:] = h_new.astype(jnp.bfloat16)

    @pl.when(t == T - 1)
    def _():
        h_out[...] = h_new
        c_out[...] = c_new


def _tc_lstm(emb, w_cat, bias):
    out_shape = [
        jax.ShapeDtypeStruct((B, HID), jnp.float32),
        jax.ShapeDtypeStruct((B, HID), jnp.float32),
    ]
    grid = (T,)
    return pl.pallas_call(
        _lstm_step,
        grid=grid,
        in_specs=[
            pl.BlockSpec((1, B, EMB), lambda t: (t, 0, 0)),
            pl.BlockSpec((EMB + HID, 4 * HID), lambda t: (0, 0)),
            pl.BlockSpec((1, 4 * HID), lambda t: (0, 0)),
        ],
        out_specs=[
            pl.BlockSpec((B, HID), lambda t: (0, 0)),
            pl.BlockSpec((B, HID), lambda t: (0, 0)),
        ],
        out_shape=out_shape,
        scratch_shapes=[
            pltpu.VMEM((B, EMB + HID), jnp.bfloat16),
            pltpu.VMEM((B, HID), jnp.float32),
        ],
    )(emb, w_cat, bias)


def kernel(x, embedding_matrix, W_ih, W_hh, b_ih, b_hh):
    idx3d = jnp.reshape(
        jnp.transpose(x).astype(jnp.int32), (_NW, _N_CHUNK, _CHUNK)
    )
    emb_flat = _sc_gather(embedding_matrix, idx3d)
    emb = jnp.reshape(emb_flat, (T, B, EMB))
    scale = jnp.concatenate([
        jnp.full((HID,), 0.5, jnp.float32),
        jnp.full((HID,), 0.5, jnp.float32),
        jnp.ones((HID,), jnp.float32),
        jnp.full((HID,), 0.5, jnp.float32),
    ])
    w_cat = (
        jnp.concatenate([jnp.transpose(W_ih), jnp.transpose(W_hh)], axis=0)
        * scale[None, :]
    ).astype(jnp.bfloat16)
    bias = jnp.reshape((b_ih + b_hh) * scale, (1, 4 * HID))
    h, c = _tc_lstm(emb, w_cat, bias)
    return (h[None, :, :], c[None, :, :])


# speedup vs baseline: 1.2116x; 1.0768x over previous
import functools

import jax
import jax.numpy as jnp
from jax import lax
from jax.experimental import pallas as pl
from jax.experimental.pallas import tpu as pltpu
from jax.experimental.pallas import tpu_sc as plsc

VOCAB = 100000
EMB = 128
HID = 256
B = 1024
T = 50

_NC = 2
_NS = 16
_NW = _NC * _NS
_N_IDX = B * T
_PER_W = _N_IDX // _NW
_CHUNK = 80
_N_CHUNK = _PER_W // _CHUNK


_NBUF = 4


def _sc_gather(table, idx2d):
    mesh = plsc.VectorSubcoreMesh(core_axis_name="c", subcore_axis_name="s")

    @functools.partial(
        pl.kernel,
        out_type=jax.ShapeDtypeStruct((_N_IDX, EMB), jnp.float32),
        mesh=mesh,
        scratch_types=[
            pltpu.VMEM((_N_CHUNK, _CHUNK), jnp.int32),
            [pltpu.VMEM((_CHUNK, EMB), jnp.float32) for _ in range(_NBUF)],
            [pltpu.SemaphoreType.DMA for _ in range(_NBUF)],
            [pltpu.SemaphoreType.DMA for _ in range(_NBUF)],
        ],
    )
    def gather_kernel(table_hbm, idx_hbm, out_hbm, idx_v, bufs, gsems, ssems):
        wid = lax.axis_index("s") * _NC + lax.axis_index("c")
        base = wid * _PER_W
        pltpu.sync_copy(idx_hbm.at[wid], idx_v)

        gathers = [None] * _N_CHUNK
        stores = [None] * _N_CHUNK

        def start_gather(j):
            b = j % _NBUF
            gathers[j] = pltpu.async_copy(
                table_hbm.at[idx_v.at[j]], bufs[b], gsems[b]
            )

        for j in range(_NBUF):
            start_gather(j)
        for j in range(_N_CHUNK):
            b = j % _NBUF
            gathers[j].wait()
            stores[j] = pltpu.async_copy(
                bufs[b], out_hbm.at[pl.ds(base + j * _CHUNK, _CHUNK)], ssems[b]
            )
            nxt = j + _NBUF
            if nxt < _N_CHUNK:
                stores[j].wait()
                start_gather(nxt)
        for j in range(_N_CHUNK - _NBUF, _N_CHUNK):
            stores[j].wait()

    return gather_kernel(table, idx2d)


_STEPS_PER_BLOCK = 2


def _lstm_step(emb_ref, w_ref, b_ref, h_out, c_out, xh_s, c_s):
    t = pl.program_id(0)

    @pl.when(t == 0)
    def _():
        xh_s[:, EMB:] = jnp.zeros((B, HID), jnp.bfloat16)
        c_s[...] = jnp.zeros_like(c_s)

    for s in range(_STEPS_PER_BLOCK):
        xh_s[:, :EMB] = emb_ref[s].astype(jnp.bfloat16)
        gates = (
            jnp.dot(xh_s[...], w_ref[...],
                    preferred_element_type=jnp.float32)
            + b_ref[...]
        )
        i = 0.5 * jnp.tanh(gates[:, 0 * HID : 1 * HID]) + 0.5
        f = 0.5 * jnp.tanh(gates[:, 1 * HID : 2 * HID]) + 0.5
        g = jnp.tanh(gates[:, 2 * HID : 3 * HID])
        o = 0.5 * jnp.tanh(gates[:, 3 * HID : 4 * HID]) + 0.5
        c_new = f * c_s[...] + i * g
        h_new = o * jnp.tanh(c_new)
        c_s[...] = c_new
        xh_s[:, EMB:] = h_new.astype(jnp.bfloat16)

    @pl.when(t == T // _STEPS_PER_BLOCK - 1)
    def _():
        h_out[...] = h_new
        c_out[...] = c_new


def _tc_lstm(emb, w_cat, bias):
    out_shape = [
        jax.ShapeDtypeStruct((B, HID), jnp.float32),
        jax.ShapeDtypeStruct((B, HID), jnp.float32),
    ]
    grid = (T // _STEPS_PER_BLOCK,)
    return pl.pallas_call(
        _lstm_step,
        grid=grid,
        in_specs=[
            pl.BlockSpec((_STEPS_PER_BLOCK, B, EMB), lambda t: (t, 0, 0)),
            pl.BlockSpec((EMB + HID, 4 * HID), lambda t: (0, 0)),
            pl.BlockSpec((1, 4 * HID), lambda t: (0, 0)),
        ],
        out_specs=[
            pl.BlockSpec((B, HID), lambda t: (0, 0)),
            pl.BlockSpec((B, HID), lambda t: (0, 0)),
        ],
        out_shape=out_shape,
        scratch_shapes=[
            pltpu.VMEM((B, EMB + HID), jnp.bfloat16),
            pltpu.VMEM((B, HID), jnp.float32),
        ],
    )(emb, w_cat, bias)


def kernel(x, embedding_matrix, W_ih, W_hh, b_ih, b_hh):
    idx3d = jnp.reshape(
        jnp.transpose(x).astype(jnp.int32), (_NW, _N_CHUNK, _CHUNK)
    )
    emb_flat = _sc_gather(embedding_matrix, idx3d)
    emb = jnp.reshape(emb_flat, (T, B, EMB))
    scale = jnp.concatenate([
        jnp.full((HID,), 0.5, jnp.float32),
        jnp.full((HID,), 0.5, jnp.float32),
        jnp.ones((HID,), jnp.float32),
        jnp.full((HID,), 0.5, jnp.float32),
    ])
    w_cat = (
        jnp.concatenate([jnp.transpose(W_ih), jnp.transpose(W_hh)], axis=0)
        * scale[None, :]
    ).astype(jnp.bfloat16)
    bias = jnp.reshape((b_ih + b_hh) * scale, (1, 4 * HID))
    h, c = _tc_lstm(emb, w_cat, bias)
    return (h[None, :, :], c[None, :, :])


# speedup vs baseline: 1.2662x; 1.0451x over previous
import functools

import jax
import jax.numpy as jnp
from jax import lax
from jax.experimental import pallas as pl
from jax.experimental.pallas import tpu as pltpu
from jax.experimental.pallas import tpu_sc as plsc

VOCAB = 100000
EMB = 128
HID = 256
B = 1024
T = 50

_NC = 2
_NS = 16
_NW = _NC * _NS
_N_IDX = B * T
_PER_W = _N_IDX // _NW
_CHUNK = 80
_N_CHUNK = _PER_W // _CHUNK


_NBUF = 4


def _sc_gather(table, idx2d):
    mesh = plsc.VectorSubcoreMesh(core_axis_name="c", subcore_axis_name="s")

    @functools.partial(
        pl.kernel,
        out_type=jax.ShapeDtypeStruct((_N_IDX, EMB), jnp.float32),
        mesh=mesh,
        scratch_types=[
            pltpu.VMEM((_N_CHUNK, _CHUNK), jnp.int32),
            [pltpu.VMEM((_CHUNK, EMB), jnp.float32) for _ in range(_NBUF)],
            [pltpu.SemaphoreType.DMA for _ in range(_NBUF)],
            [pltpu.SemaphoreType.DMA for _ in range(_NBUF)],
        ],
    )
    def gather_kernel(table_hbm, idx_hbm, out_hbm, idx_v, bufs, gsems, ssems):
        wid = lax.axis_index("s") * _NC + lax.axis_index("c")
        base = wid * _PER_W
        pltpu.sync_copy(idx_hbm.at[wid], idx_v)

        gathers = [None] * _N_CHUNK
        stores = [None] * _N_CHUNK

        def start_gather(j):
            b = j % _NBUF
            gathers[j] = pltpu.async_copy(
                table_hbm.at[idx_v.at[j]], bufs[b], gsems[b]
            )

        for j in range(_NBUF):
            start_gather(j)
        for j in range(_N_CHUNK):
            b = j % _NBUF
            gathers[j].wait()
            stores[j] = pltpu.async_copy(
                bufs[b], out_hbm.at[pl.ds(base + j * _CHUNK, _CHUNK)], ssems[b]
            )
            nxt = j + _NBUF
            if nxt < _N_CHUNK:
                stores[j].wait()
                start_gather(nxt)
        for j in range(_N_CHUNK - _NBUF, _N_CHUNK):
            stores[j].wait()

    return gather_kernel(table, idx2d)


_STEPS_PER_BLOCK = 5


def _lstm_step(emb_ref, w_ref, b_ref, h_out, c_out, xh_s, c_s):
    t = pl.program_id(0)

    @pl.when(t == 0)
    def _():
        xh_s[:, EMB:] = jnp.zeros((B, HID), jnp.bfloat16)
        c_s[...] = jnp.zeros_like(c_s)

    for s in range(_STEPS_PER_BLOCK):
        xh_s[:, :EMB] = emb_ref[s].astype(jnp.bfloat16)
        gates = (
            jnp.dot(xh_s[...], w_ref[...],
                    preferred_element_type=jnp.float32)
            + b_ref[...]
        )
        i = 0.5 * jnp.tanh(gates[:, 0 * HID : 1 * HID]) + 0.5
        f = 0.5 * jnp.tanh(gates[:, 1 * HID : 2 * HID]) + 0.5
        g = jnp.tanh(gates[:, 2 * HID : 3 * HID])
        o = 0.5 * jnp.tanh(gates[:, 3 * HID : 4 * HID]) + 0.5
        c_new = f * c_s[...] + i * g
        h_new = o * jnp.tanh(c_new)
        c_s[...] = c_new
        xh_s[:, EMB:] = h_new.astype(jnp.bfloat16)

    @pl.when(t == T // _STEPS_PER_BLOCK - 1)
    def _():
        h_out[...] = h_new
        c_out[...] = c_new


def _tc_lstm(emb, w_cat, bias):
    out_shape = [
        jax.ShapeDtypeStruct((B, HID), jnp.float32),
        jax.ShapeDtypeStruct((B, HID), jnp.float32),
    ]
    grid = (T // _STEPS_PER_BLOCK,)
    return pl.pallas_call(
        _lstm_step,
        grid=grid,
        in_specs=[
            pl.BlockSpec((_STEPS_PER_BLOCK, B, EMB), lambda t: (t, 0, 0)),
            pl.BlockSpec((EMB + HID, 4 * HID), lambda t: (0, 0)),
            pl.BlockSpec((1, 4 * HID), lambda t: (0, 0)),
        ],
        out_specs=[
            pl.BlockSpec((B, HID), lambda t: (0, 0)),
            pl.BlockSpec((B, HID), lambda t: (0, 0)),
        ],
        out_shape=out_shape,
        scratch_shapes=[
            pltpu.VMEM((B, EMB + HID), jnp.bfloat16),
            pltpu.VMEM((B, HID), jnp.float32),
        ],
    )(emb, w_cat, bias)


def kernel(x, embedding_matrix, W_ih, W_hh, b_ih, b_hh):
    idx3d = jnp.reshape(
        jnp.transpose(x).astype(jnp.int32), (_NW, _N_CHUNK, _CHUNK)
    )
    emb_flat = _sc_gather(embedding_matrix, idx3d)
    emb = jnp.reshape(emb_flat, (T, B, EMB))
    scale = jnp.concatenate([
        jnp.full((HID,), 0.5, jnp.float32),
        jnp.full((HID,), 0.5, jnp.float32),
        jnp.ones((HID,), jnp.float32),
        jnp.full((HID,), 0.5, jnp.float32),
    ])
    w_cat = (
        jnp.concatenate([jnp.transpose(W_ih), jnp.transpose(W_hh)], axis=0)
        * scale[None, :]
    ).astype(jnp.bfloat16)
    bias = jnp.reshape((b_ih + b_hh) * scale, (1, 4 * HID))
    h, c = _tc_lstm(emb, w_cat, bias)
    return (h[None, :, :], c[None, :, :])
